# baseline (speedup 1.0000x reference)
import functools

import jax
import jax.numpy as jnp
from jax import lax
from jax.experimental import pallas as pl
from jax.experimental.pallas import tpu as pltpu
from jax.experimental.pallas import tpu_sc as plsc

VOCAB = 1000000
EMBED = 64
SEQ = 50
BATCH = 4096

NC = 2
NS = 16
NW = NC * NS

SEQ_W = BATCH // NW
SPC = 4
CHUNK = SPC * SEQ
N_CHUNKS = SEQ_W // SPC
RING = 8
LOOK = 5
N_OUTER = N_CHUNKS // RING


def _make_sc_kernel():
    mesh = plsc.VectorSubcoreMesh(core_axis_name="c", subcore_axis_name="s")

    @functools.partial(
        pl.kernel,
        mesh=mesh,
        out_type=jax.ShapeDtypeStruct((BATCH, SEQ, EMBED), jnp.float32),
        compiler_params=pltpu.CompilerParams(use_tc_tiling_on_sc=False),
        scratch_types=[
            pltpu.VMEM((SEQ_W, SEQ), jnp.int32),
            pltpu.VMEM((SEQ_W, 64), jnp.float32),
            pltpu.VMEM((RING, SPC, SEQ, EMBED), jnp.float32),
            pltpu.VMEM((SEQ, EMBED), jnp.float32),
        ] + [pltpu.SemaphoreType.DMA] * (2 * RING),
    )
    def sc_kernel(tok_hbm, maskp_hbm, table_hbm, pos_hbm, out_hbm,
                  idx_v, maskp_v, rows_v, pos_v, *sems):
        gsem = sems[:RING]
        wsem = sems[RING:]
        wid = lax.axis_index("s") * NC + lax.axis_index("c")
        srow_base = pl.multiple_of(wid * SEQ_W, SEQ_W)

        pltpu.sync_copy(pos_hbm, pos_v)
        pltpu.sync_copy(tok_hbm.at[pl.ds(srow_base, SEQ_W)], idx_v)
        pltpu.sync_copy(maskp_hbm.at[pl.ds(srow_base, SEQ_W)], maskp_v)

        lanes = lax.broadcasted_iota(jnp.int32, (16,), 0)
        dnums = lax.GatherDimensionNumbers(
            offset_dims=(), collapsed_slice_dims=(0,), start_index_map=(0,))

        def fire_gather(b, ci):
            for g in range(SPC):
                pltpu.async_copy(
                    table_hbm.at[idx_v.at[ci * SPC + g]],
                    rows_v.at[b, g], gsem[b])

        def wait_gather(b, ci):
            for g in range(SPC):
                pltpu.make_async_copy(
                    table_hbm.at[idx_v.at[ci * SPC + g]],
                    rows_v.at[b, g], gsem[b]).wait()

        def fire_wb(b, ci):
            orow = pl.multiple_of(srow_base + ci * SPC, SPC)
            pltpu.async_copy(
                rows_v.at[b], out_hbm.at[pl.ds(orow, SPC)], wsem[b])

        def wait_wb(b):
            pltpu.make_async_copy(
                rows_v.at[b], out_hbm.at[pl.ds(srow_base, SPC)],
                wsem[b]).wait()

        def compute(b, ci):
            srow0 = ci * SPC

            def s_body(s, c2):
                k16 = (s // 16) * 16
                lane = s % 16
                gidx = lanes * 0 + lane
                prow = [pos_v[s, pl.ds(dg * 16, 16)] for dg in range(4)]
                for si in range(SPC):
                    mrow = maskp_v[srow0 + si, pl.ds(k16, 16)]
                    m = lax.gather(
                        mrow, gidx[:, None], dnums, (1,),
                        mode=lax.GatherScatterMode.PROMISE_IN_BOUNDS)
                    for dg in range(EMBED // 16):
                        sl = pl.ds(dg * 16, 16)
                        rows_v[b, si, s, sl] = (rows_v[b, si, s, sl]
                                                + prow[dg]) * m
                return c2

            lax.fori_loop(0, SEQ, s_body, 0)

        for b0 in range(LOOK):
            fire_gather(b0, b0)

        def outer_body(p, carry):
            for j in range(RING):
                ci = p * RING + j
                fb = (j + LOOK) % RING
                fci = ci + LOOK

                @pl.when(jnp.logical_and(fci >= RING, fci < N_CHUNKS))
                def _():
                    wait_wb(fb)

                @pl.when(fci < N_CHUNKS)
                def _():
                    fire_gather(fb, fci)

                wait_gather(j, ci)
                compute(j, ci)
                fire_wb(j, ci)
            return carry

        lax.fori_loop(0, N_OUTER, outer_body, 0)
        for b in range(RING):
            wait_wb(b)

    return sc_kernel


_SC_KERNEL = _make_sc_kernel()


@jax.jit
def kernel(tokens, mask, token_embed, pos_embed):
    maskp = jnp.pad(mask, ((0, 0), (0, 64 - SEQ)))
    pos = pos_embed.reshape(SEQ, EMBED)
    return _SC_KERNEL(tokens.astype(jnp.int32), maskp, token_embed, pos)
